# baseline (speedup 1.0000x reference)
import jax
import jax.numpy as jnp
from jax.experimental import pallas as pl
from jax.experimental.pallas import tpu as pltpu

_BLOCK_ROWS = 256


def _diag_add_body(x_ref, s_ref, o_ref):
    i = pl.program_id(0)
    blk = x_ref[...]
    rows, cols = blk.shape
    r = jax.lax.broadcasted_iota(jnp.int32, (rows, cols), 0)
    c = jax.lax.broadcasted_iota(jnp.int32, (rows, cols), 1)
    mask = c == r + i * rows
    o_ref[...] = blk + jnp.where(mask, s_ref[0], jnp.float32(0.0))


def kernel(x, sigma2):
    n, m = x.shape
    br = _BLOCK_ROWS if n % _BLOCK_ROWS == 0 else n
    grid = (n // br,)
    return pl.pallas_call(
        _diag_add_body,
        grid=grid,
        in_specs=[
            pl.BlockSpec((br, m), lambda i: (i, 0)),
            pl.BlockSpec(memory_space=pltpu.SMEM),
        ],
        out_specs=pl.BlockSpec((br, m), lambda i: (i, 0)),
        out_shape=jax.ShapeDtypeStruct((n, m), x.dtype),
        compiler_params=pltpu.CompilerParams(
            dimension_semantics=("parallel",),
            vmem_limit_bytes=100 * 1024 * 1024,
        ),
    )(x, sigma2)
